# baseline (speedup 1.0000x reference)
import jax
import jax.numpy as jnp
import numpy as np
from jax.experimental import pallas as pl


def _fps_single(xyz, npoint):
    n = xyz.shape[0]

    def body(i, state):
        dists, farthest, inds = state
        inds = inds.at[i].set(farthest)
        d = jnp.sum((xyz - xyz[farthest]) ** 2, axis=-1)
        dists = jnp.minimum(dists, d)
        farthest = jnp.argmax(dists).astype(jnp.int32)
        return (dists, farthest, inds)

    dists = jnp.full((n,), 1e10, dtype=xyz.dtype)
    inds0 = jnp.zeros((npoint,), dtype=jnp.int32)
    _, _, inds = jax.lax.fori_loop(0, npoint, body, (dists, jnp.int32(0), inds0))
    return inds


def _ball_query_single(radius, nsample, xyz, new_xyz):
    n = xyz.shape[0]
    sqr = jnp.sum((new_xyz[:, None, :] - xyz[None, :, :]) ** 2, axis=-1)
    cand = jnp.where(sqr < radius * radius,
                     jnp.arange(n, dtype=jnp.int32)[None, :], jnp.int32(n))
    neg_vals, _ = jax.lax.top_k(-cand, nsample)
    idx = -neg_vals
    first = idx[:, :1]
    idx = jnp.where(idx == n, first, idx)
    return idx


def _shared_mlp(x, layers):
    for lyr in layers:
        x = jnp.einsum('oc,bcns->bons', lyr['W'], x)
        mean = jnp.mean(x, axis=(0, 2, 3), keepdims=True)
        var = jnp.var(x, axis=(0, 2, 3), keepdims=True)
        x = (x - mean) / jnp.sqrt(var + 1e-5)
        x = x * lyr['gamma'][None, :, None, None] + lyr['beta'][None, :, None, None]
        x = jax.nn.relu(x)
    return x


def _sa_module(xyz, features, npoint, radius, nsample, layers):
    inds = jax.vmap(lambda p: _fps_single(p, npoint))(xyz)
    new_xyz = jax.vmap(lambda p, i: p[i])(xyz, inds)
    idx = jax.vmap(lambda p, q: _ball_query_single(radius, nsample, p, q))(xyz, new_xyz)
    grouped_xyz = jax.vmap(lambda p, i: p[i])(xyz, idx)
    grouped_xyz = (grouped_xyz - new_xyz[:, :, None, :]) / radius
    grouped_xyz = jnp.transpose(grouped_xyz, (0, 3, 1, 2))
    if features is not None:
        grouped_feats = jax.vmap(lambda f, i: f[:, i])(features, idx)
        grouped = jnp.concatenate([grouped_xyz, grouped_feats], axis=1)
    else:
        grouped = grouped_xyz
    new_features = _shared_mlp(grouped, layers)
    new_features = jnp.max(new_features, axis=-1)
    return new_xyz, new_features, inds


def _fp_module(unknown, known, unknow_feats, known_feats, layers):
    sqr = jnp.sum((unknown[:, :, None, :] - known[:, None, :, :]) ** 2, axis=-1)
    neg, idx = jax.lax.top_k(-sqr, 3)
    dist = -neg
    dist_recip = 1.0 / (dist + 1e-8)
    norm = jnp.sum(dist_recip, axis=2, keepdims=True)
    weight = dist_recip / norm
    gathered = jax.vmap(lambda f, i: f[:, i])(known_feats, idx)
    interpolated = jnp.sum(gathered * weight[:, None, :, :], axis=-1)
    new_features = jnp.concatenate([interpolated, unknow_feats], axis=1)
    return _shared_mlp(new_features[..., None], layers)[..., 0]


def _backbone_impl(pointcloud, params):
    xyz = pointcloud[..., 0:3]
    sa1_xyz, sa1_f, sa1_inds = _sa_module(xyz, None, 2048, 0.2, 64, params['sa1'])
    sa2_xyz, sa2_f, _ = _sa_module(sa1_xyz, sa1_f, 1024, 0.4, 32, params['sa2'])
    sa3_xyz, sa3_f, _ = _sa_module(sa2_xyz, sa2_f, 512, 0.8, 16, params['sa3'])
    sa4_xyz, sa4_f, _ = _sa_module(sa3_xyz, sa3_f, 256, 1.2, 16, params['sa4'])
    f = _fp_module(sa3_xyz, sa4_xyz, sa3_f, sa4_f, params['fp1'])
    f = _fp_module(sa2_xyz, sa3_xyz, sa2_f, f, params['fp2'])
    return f, sa2_xyz, sa1_inds[:, :1024]


_backbone_jitted = jax.jit(_backbone_impl)


def kernel(pointcloud, params):
    return _backbone_jitted(pointcloud, params)


# speedup vs baseline: 1.7689x; 1.7689x over previous
import functools

import jax
import jax.numpy as jnp
import numpy as np
from jax import lax
from jax.experimental import pallas as pl
from jax.experimental.pallas import tpu as pltpu


def _fps_body(xyz_ref, inds_ref, newx_ref, dists_ref, *, npoint, n):
    n8 = n // 8
    x = xyz_ref[0, 0, :, :]
    y = xyz_ref[0, 1, :, :]
    z = xyz_ref[0, 2, :, :]
    jgrid = (lax.broadcasted_iota(jnp.int32, (8, n8), 0) * n8
             + lax.broadcasted_iota(jnp.int32, (8, n8), 1))
    dists_ref[:, :] = jnp.full((8, n8), 1e10, dtype=jnp.float32)

    def body(i, farthest):
        fsel = jgrid == farthest
        fx = jnp.sum(jnp.where(fsel, x, 0.0))
        fy = jnp.sum(jnp.where(fsel, y, 0.0))
        fz = jnp.sum(jnp.where(fsel, z, 0.0))
        inds_ref[0, 0, i] = farthest
        newx_ref[0, 0, i] = fx
        newx_ref[0, 1, i] = fy
        newx_ref[0, 2, i] = fz
        dx = x - fx
        dy = y - fy
        dz = z - fz
        d = dx * dx + dy * dy + dz * dz
        dn = jnp.minimum(dists_ref[:, :], d)
        dists_ref[:, :] = dn
        m = jnp.max(dn)
        fnew = jnp.min(jnp.where(dn == m, jgrid, n)).astype(jnp.int32)
        return fnew

    lax.fori_loop(0, npoint, body, jnp.int32(0))


def _fps_pallas(xyzT, npoint):
    B, _, n = xyzT.shape
    n8 = n // 8
    xyz4 = xyzT.reshape(B, 3, 8, n8)
    inds, newx = pl.pallas_call(
        functools.partial(_fps_body, npoint=npoint, n=n),
        grid=(B,),
        in_specs=[pl.BlockSpec((1, 3, 8, n8), lambda b: (b, 0, 0, 0))],
        out_specs=[
            pl.BlockSpec((1, 1, npoint), lambda b: (b, 0, 0),
                         memory_space=pltpu.SMEM),
            pl.BlockSpec((1, 3, npoint), lambda b: (b, 0, 0),
                         memory_space=pltpu.SMEM),
        ],
        out_shape=[
            jax.ShapeDtypeStruct((B, 1, npoint), jnp.int32),
            jax.ShapeDtypeStruct((B, 3, npoint), jnp.float32),
        ],
        scratch_shapes=[pltpu.VMEM((8, n8), jnp.float32)],
    )(xyz4)
    return inds[:, 0], jnp.transpose(newx, (0, 2, 1))


def _fps_single(xyz, npoint):
    n = xyz.shape[0]

    def body(i, state):
        dists, farthest, inds = state
        inds = inds.at[i].set(farthest)
        d = jnp.sum((xyz - xyz[farthest]) ** 2, axis=-1)
        dists = jnp.minimum(dists, d)
        farthest = jnp.argmax(dists).astype(jnp.int32)
        return (dists, farthest, inds)

    dists = jnp.full((n,), 1e10, dtype=xyz.dtype)
    inds0 = jnp.zeros((npoint,), dtype=jnp.int32)
    _, _, inds = jax.lax.fori_loop(0, npoint, body, (dists, jnp.int32(0), inds0))
    return inds


def _ball_query_single(radius, nsample, xyz, new_xyz):
    n = xyz.shape[0]
    sqr = jnp.sum((new_xyz[:, None, :] - xyz[None, :, :]) ** 2, axis=-1)
    cand = jnp.where(sqr < radius * radius,
                     jnp.arange(n, dtype=jnp.int32)[None, :], jnp.int32(n))
    neg_vals, _ = jax.lax.top_k(-cand, nsample)
    idx = -neg_vals
    first = idx[:, :1]
    idx = jnp.where(idx == n, first, idx)
    return idx


def _shared_mlp(x, layers):
    for lyr in layers:
        x = jnp.einsum('oc,bcns->bons', lyr['W'], x)
        mean = jnp.mean(x, axis=(0, 2, 3), keepdims=True)
        var = jnp.var(x, axis=(0, 2, 3), keepdims=True)
        x = (x - mean) / jnp.sqrt(var + 1e-5)
        x = x * lyr['gamma'][None, :, None, None] + lyr['beta'][None, :, None, None]
        x = jax.nn.relu(x)
    return x


def _sa_module(xyz, features, npoint, radius, nsample, layers):
    inds, new_xyz = _fps_pallas(jnp.transpose(xyz, (0, 2, 1)), npoint)
    idx = jax.vmap(lambda p, q: _ball_query_single(radius, nsample, p, q))(xyz, new_xyz)
    grouped_xyz = jax.vmap(lambda p, i: p[i])(xyz, idx)
    grouped_xyz = (grouped_xyz - new_xyz[:, :, None, :]) / radius
    grouped_xyz = jnp.transpose(grouped_xyz, (0, 3, 1, 2))
    if features is not None:
        grouped_feats = jax.vmap(lambda f, i: f[:, i])(features, idx)
        grouped = jnp.concatenate([grouped_xyz, grouped_feats], axis=1)
    else:
        grouped = grouped_xyz
    new_features = _shared_mlp(grouped, layers)
    new_features = jnp.max(new_features, axis=-1)
    return new_xyz, new_features, inds


def _fp_module(unknown, known, unknow_feats, known_feats, layers):
    sqr = jnp.sum((unknown[:, :, None, :] - known[:, None, :, :]) ** 2, axis=-1)
    neg, idx = jax.lax.top_k(-sqr, 3)
    dist = -neg
    dist_recip = 1.0 / (dist + 1e-8)
    norm = jnp.sum(dist_recip, axis=2, keepdims=True)
    weight = dist_recip / norm
    gathered = jax.vmap(lambda f, i: f[:, i])(known_feats, idx)
    interpolated = jnp.sum(gathered * weight[:, None, :, :], axis=-1)
    new_features = jnp.concatenate([interpolated, unknow_feats], axis=1)
    return _shared_mlp(new_features[..., None], layers)[..., 0]


def _backbone_impl(pointcloud, params):
    xyz = pointcloud[..., 0:3]
    sa1_xyz, sa1_f, sa1_inds = _sa_module(xyz, None, 2048, 0.2, 64, params['sa1'])
    sa2_xyz, sa2_f, _ = _sa_module(sa1_xyz, sa1_f, 1024, 0.4, 32, params['sa2'])
    sa3_xyz, sa3_f, _ = _sa_module(sa2_xyz, sa2_f, 512, 0.8, 16, params['sa3'])
    sa4_xyz, sa4_f, _ = _sa_module(sa3_xyz, sa3_f, 256, 1.2, 16, params['sa4'])
    f = _fp_module(sa3_xyz, sa4_xyz, sa3_f, sa4_f, params['fp1'])
    f = _fp_module(sa2_xyz, sa3_xyz, sa2_f, f, params['fp2'])
    return f, sa2_xyz, sa1_inds[:, :1024]


_backbone_jitted = jax.jit(_backbone_impl)


def kernel(pointcloud, params):
    return _backbone_jitted(pointcloud, params)


# speedup vs baseline: 1.7788x; 1.0056x over previous
import functools

import jax
import jax.numpy as jnp
import numpy as np
from jax import lax
from jax.experimental import pallas as pl
from jax.experimental.pallas import tpu as pltpu


def _fps_body(xyz_ref, inds_ref, newx_ref, dists_ref, *, npoint, n):
    n8 = n // 8
    x = xyz_ref[0, 0, :, :]
    y = xyz_ref[0, 1, :, :]
    z = xyz_ref[0, 2, :, :]
    jgrid = (lax.broadcasted_iota(jnp.int32, (8, n8), 0) * n8
             + lax.broadcasted_iota(jnp.int32, (8, n8), 1))
    dists_ref[:, :] = jnp.full((8, n8), 1e10, dtype=jnp.float32)

    def body(i, farthest):
        fsel = jgrid == farthest
        fx = jnp.sum(jnp.where(fsel, x, 0.0))
        fy = jnp.sum(jnp.where(fsel, y, 0.0))
        fz = jnp.sum(jnp.where(fsel, z, 0.0))
        inds_ref[0, 0, i] = farthest
        newx_ref[0, 0, i] = fx
        newx_ref[0, 1, i] = fy
        newx_ref[0, 2, i] = fz
        dx = x - fx
        dy = y - fy
        dz = z - fz
        d = dx * dx + dy * dy + dz * dz
        dn = jnp.minimum(dists_ref[:, :], d)
        dists_ref[:, :] = dn
        m = jnp.max(dn)
        fnew = jnp.min(jnp.where(dn == m, jgrid, n)).astype(jnp.int32)
        return fnew

    lax.fori_loop(0, npoint, body, jnp.int32(0))


def _fps_pallas(xyzT, npoint):
    B, _, n = xyzT.shape
    n8 = n // 8
    xyz4 = xyzT.reshape(B, 3, 8, n8)
    inds, newx = pl.pallas_call(
        functools.partial(_fps_body, npoint=npoint, n=n),
        grid=(B,),
        in_specs=[pl.BlockSpec((1, 3, 8, n8), lambda b: (b, 0, 0, 0))],
        out_specs=[
            pl.BlockSpec((1, 1, npoint), lambda b: (b, 0, 0),
                         memory_space=pltpu.SMEM),
            pl.BlockSpec((1, 3, npoint), lambda b: (b, 0, 0),
                         memory_space=pltpu.SMEM),
        ],
        out_shape=[
            jax.ShapeDtypeStruct((B, 1, npoint), jnp.int32),
            jax.ShapeDtypeStruct((B, 3, npoint), jnp.float32),
        ],
        scratch_shapes=[pltpu.VMEM((8, n8), jnp.float32)],
    )(xyz4)
    return inds[:, 0], jnp.transpose(newx, (0, 2, 1))


def _dense_copy(x):
    m = x.shape[0]
    r = m // 128
    if r <= 512:
        d = r
    else:
        d = next(dd for dd in (512, 256, 128, 64, 32, 16, 8) if r % dd == 0)

    def body(i_ref, o_ref):
        o_ref[...] = i_ref[...]

    out = pl.pallas_call(
        body,
        grid=(r // d,),
        in_specs=[pl.BlockSpec((d, 128), lambda i: (i, 0))],
        out_specs=pl.BlockSpec((d, 128), lambda i: (i, 0)),
        out_shape=jax.ShapeDtypeStruct((r, 128), x.dtype),
    )(x.reshape(r, 128))
    return out.reshape(m)


def _ball_query_sc(xyz, cent, radius, nsample):
    from jax.experimental.pallas import tpu_sc as plsc

    B, n, _ = xyz.shape
    NP = cent.shape[1]
    NS = nsample
    NW = 32
    WPB = NW // B
    NPC = NP // WPB
    U = 8
    NCH = n // 16
    NBLK = NCH // U
    SLACK = NS + 16 * U
    r2 = float(radius) * float(radius)
    radius = float(radius)

    mesh = plsc.VectorSubcoreMesh(core_axis_name="c", subcore_axis_name="s")

    @functools.partial(
        pl.kernel,
        mesh=mesh,
        out_type=[
            jax.ShapeDtypeStruct((B * NP * NS,), jnp.int32),
            jax.ShapeDtypeStruct((B * NP * NS * 8,), jnp.float32),
        ],
        scratch_types=[
            pltpu.VMEM((3 * n,), jnp.float32),
            pltpu.VMEM((n,), jnp.float32),
            pltpu.VMEM((n,), jnp.float32),
            pltpu.VMEM((n,), jnp.float32),
            pltpu.VMEM((NP,), jnp.float32),
            pltpu.VMEM((NP,), jnp.float32),
            pltpu.VMEM((NP,), jnp.float32),
            pltpu.VMEM((SLACK,), jnp.int32),
            pltpu.VMEM((NS * 8,), jnp.float32),
        ],
        compiler_params=pltpu.CompilerParams(needs_layout_passes=False),
    )
    def bq(xyz_hbm, cent_hbm, idx_hbm, gx_hbm, pb, xb, yb, zb, cxb, cyb, czb,
           idxb, gb):
        wid = lax.axis_index("s") * 2 + lax.axis_index("c")
        b = wid // WPB
        t = wid % WPB
        c0 = t * NPC
        iota = lax.iota(jnp.int32, 16)

        pltpu.sync_copy(cent_hbm.at[pl.ds(b * 3 * NP, 3 * NP)],
                        pb.at[pl.ds(0, 3 * NP)])

        def deint_c(j, carry):
            srcv = (iota + j * 16) * 3
            cxb[pl.ds(j * 16, 16)] = plsc.load_gather(pb, [srcv])
            cyb[pl.ds(j * 16, 16)] = plsc.load_gather(pb, [srcv + 1])
            czb[pl.ds(j * 16, 16)] = plsc.load_gather(pb, [srcv + 2])
            return carry

        lax.fori_loop(0, NP // 16, deint_c, jnp.int32(0))

        pltpu.sync_copy(xyz_hbm.at[pl.ds(b * 3 * n, 3 * n)], pb)

        def deint_p(j, carry):
            srcv = (iota + j * 16) * 3
            xb[pl.ds(j * 16, 16)] = plsc.load_gather(pb, [srcv])
            yb[pl.ds(j * 16, 16)] = plsc.load_gather(pb, [srcv + 1])
            zb[pl.ds(j * 16, 16)] = plsc.load_gather(pb, [srcv + 2])
            return carry

        lax.fori_loop(0, n // 16, deint_p, jnp.int32(0))

        for g in range(NS * 8 // 16):
            gb[pl.ds(g * 16, 16)] = jnp.zeros((16,), jnp.float32)

        def per_centroid(k, carry):
            k16 = jnp.full((16,), c0 + k, jnp.int32)
            cx = plsc.load_gather(cxb, [k16])
            cy = plsc.load_gather(cyb, [k16])
            cz = plsc.load_gather(czb, [k16])

            def cond(st):
                blk, cur = st
                return jnp.logical_and(blk < NBLK, cur < NS)

            def sbody(st):
                blk, cur = st
                base0 = blk * (16 * U)
                masks = []
                cnts = []
                for u in range(U):
                    off = base0 + u * 16
                    px = xb[pl.ds(off, 16)]
                    py = yb[pl.ds(off, 16)]
                    pz = zb[pl.ds(off, 16)]
                    dx = px - cx
                    dy = py - cy
                    dz = pz - cz
                    d = dx * dx + dy * dy + dz * dz
                    m = d < r2
                    masks.append(m)
                    cnts.append(jnp.sum(m.astype(jnp.int32)))
                for u in range(U):
                    prefix = plsc.cumsum(masks[u].astype(jnp.int32))
                    pos = jnp.maximum(cur + prefix - 1, 0)
                    plsc.store_scatter(idxb, [pos],
                                       iota + (base0 + u * 16),
                                       mask=masks[u])
                    cur = cur + cnts[u]
                return (blk + jnp.int32(1), cur)

            _, cursor = lax.while_loop(cond, sbody,
                                       (jnp.int32(0), jnp.int32(0)))

            first = plsc.load_gather(idxb, [jnp.zeros((16,), jnp.int32)])
            for g in range(NS // 16):
                pos = iota + g * 16
                curv = idxb[pl.ds(g * 16, 16)]
                idxb[pl.ds(g * 16, 16)] = jnp.where(pos < cursor, curv, first)

            for g in range(NS // 16):
                iv = idxb[pl.ds(g * 16, 16)]
                gx = (plsc.load_gather(xb, [iv]) - cx) / radius
                gy = (plsc.load_gather(yb, [iv]) - cy) / radius
                gz = (plsc.load_gather(zb, [iv]) - cz) / radius
                lp = (iota + g * 16) * 8
                plsc.store_scatter(gb, [lp], gx)
                plsc.store_scatter(gb, [lp + 1], gy)
                plsc.store_scatter(gb, [lp + 2], gz)

            ci = (b * NP + c0 + k) * NS
            pltpu.sync_copy(idxb.at[pl.ds(0, NS)], idx_hbm.at[pl.ds(ci, NS)])
            pltpu.sync_copy(gb, gx_hbm.at[pl.ds(ci * 8, NS * 8)])
            return carry

        lax.fori_loop(0, NPC, per_centroid, jnp.int32(0))

    idx_flat, gx = bq(_dense_copy(xyz.reshape(-1)),
                      _dense_copy(cent.reshape(-1)))
    idx = _dense_copy(idx_flat).reshape(B, NP, NS)
    gx = _dense_copy(gx).reshape(B, NP * NS, 8)
    return idx, gx


def _fps_single(xyz, npoint):
    n = xyz.shape[0]

    def body(i, state):
        dists, farthest, inds = state
        inds = inds.at[i].set(farthest)
        d = jnp.sum((xyz - xyz[farthest]) ** 2, axis=-1)
        dists = jnp.minimum(dists, d)
        farthest = jnp.argmax(dists).astype(jnp.int32)
        return (dists, farthest, inds)

    dists = jnp.full((n,), 1e10, dtype=xyz.dtype)
    inds0 = jnp.zeros((npoint,), dtype=jnp.int32)
    _, _, inds = jax.lax.fori_loop(0, npoint, body, (dists, jnp.int32(0), inds0))
    return inds


def _ball_query_single(radius, nsample, xyz, new_xyz):
    n = xyz.shape[0]
    sqr = jnp.sum((new_xyz[:, None, :] - xyz[None, :, :]) ** 2, axis=-1)
    cand = jnp.where(sqr < radius * radius,
                     jnp.arange(n, dtype=jnp.int32)[None, :], jnp.int32(n))
    neg_vals, _ = jax.lax.top_k(-cand, nsample)
    idx = -neg_vals
    first = idx[:, :1]
    idx = jnp.where(idx == n, first, idx)
    return idx


def _shared_mlp(x, layers):
    for lyr in layers:
        x = jnp.einsum('oc,bcns->bons', lyr['W'], x)
        mean = jnp.mean(x, axis=(0, 2, 3), keepdims=True)
        var = jnp.var(x, axis=(0, 2, 3), keepdims=True)
        x = (x - mean) / jnp.sqrt(var + 1e-5)
        x = x * lyr['gamma'][None, :, None, None] + lyr['beta'][None, :, None, None]
        x = jax.nn.relu(x)
    return x


def _sa_module(xyz, features, npoint, radius, nsample, layers):
    B = xyz.shape[0]
    xyzT = jnp.transpose(xyz, (0, 2, 1))
    inds, new_xyz = _fps_pallas(xyzT, npoint)
    idx = jax.vmap(lambda p, q: _ball_query_single(radius, nsample, p, q))(
        xyz, new_xyz)
    grouped_xyz = jax.vmap(lambda p, i: p[i])(xyz, idx)
    grouped_xyz = (grouped_xyz - new_xyz[:, :, None, :]) / radius
    grouped_xyz = jnp.transpose(grouped_xyz, (0, 3, 1, 2))
    if features is not None:
        grouped_feats = jax.vmap(lambda f, i: f[:, i])(features, idx)
        grouped = jnp.concatenate([grouped_xyz, grouped_feats], axis=1)
    else:
        grouped = grouped_xyz
    new_features = _shared_mlp(grouped, layers)
    new_features = jnp.max(new_features, axis=-1)
    return new_xyz, new_features, inds


def _fp_module(unknown, known, unknow_feats, known_feats, layers):
    sqr = jnp.sum((unknown[:, :, None, :] - known[:, None, :, :]) ** 2, axis=-1)
    neg, idx = jax.lax.top_k(-sqr, 3)
    dist = -neg
    dist_recip = 1.0 / (dist + 1e-8)
    norm = jnp.sum(dist_recip, axis=2, keepdims=True)
    weight = dist_recip / norm
    gathered = jax.vmap(lambda f, i: f[:, i])(known_feats, idx)
    interpolated = jnp.sum(gathered * weight[:, None, :, :], axis=-1)
    new_features = jnp.concatenate([interpolated, unknow_feats], axis=1)
    return _shared_mlp(new_features[..., None], layers)[..., 0]


def _backbone_impl(pointcloud, params):
    xyz = pointcloud[..., 0:3]
    sa1_xyz, sa1_f, sa1_inds = _sa_module(xyz, None, 2048, 0.2, 64, params['sa1'])
    sa2_xyz, sa2_f, _ = _sa_module(sa1_xyz, sa1_f, 1024, 0.4, 32, params['sa2'])
    sa3_xyz, sa3_f, _ = _sa_module(sa2_xyz, sa2_f, 512, 0.8, 16, params['sa3'])
    sa4_xyz, sa4_f, _ = _sa_module(sa3_xyz, sa3_f, 256, 1.2, 16, params['sa4'])
    f = _fp_module(sa3_xyz, sa4_xyz, sa3_f, sa4_f, params['fp1'])
    f = _fp_module(sa2_xyz, sa3_xyz, sa2_f, f, params['fp2'])
    return f, sa2_xyz, sa1_inds[:, :1024]


_backbone_jitted = jax.jit(_backbone_impl)


def kernel(pointcloud, params):
    return _backbone_jitted(pointcloud, params)
